# baseline (speedup 1.0000x reference)
import jax
import jax.numpy as jnp
from jax import lax
from jax.experimental import pallas as pl
from jax.experimental.pallas import tpu as pltpu
from jax.experimental.pallas import tpu_sc as plsc

_NC = 2
_NS = 16
_NW = _NC * _NS
_L = 16

_N = 16384
_F = 2
_B = 128
_NB = _N // _B
_BLKS_PER = _NB // _NW
_ROWS_PER = _BLKS_PER * _B
_CHUNKS = _B // _L


def _body(x_hbm, labels_hbm, cm_hbm, out_hbm, x_v, lab_v, cm_v, sem):
    wid = lax.axis_index("s") * _NC + lax.axis_index("c")
    bbase = wid * _BLKS_PER
    rbase = wid * _ROWS_PER

    cx = pltpu.async_copy(x_hbm.at[pl.ds(bbase, _BLKS_PER)], x_v, sem)
    cl = pltpu.async_copy(labels_hbm.at[pl.ds(rbase, _ROWS_PER)], lab_v, sem)
    cc = pltpu.async_copy(cm_hbm, cm_v, sem)
    cx.wait()
    cl.wait()
    cc.wait()

    for b in range(_BLKS_PER):
        for f in range(_F):
            for c in range(_CHUNKS):
                rlo = b * _B + c * _L
                lab = lab_v[pl.ds(rlo, _L)]
                g = plsc.load_gather(cm_v, [lab + 3 * f])
                x_v[b, f, pl.ds(c * _L, _L)] -= g

    pltpu.sync_copy(x_v, out_hbm.at[pl.ds(bbase, _BLKS_PER)])


_sc_call = pl.kernel(
    _body,
    out_type=jax.ShapeDtypeStruct((_NB, _F, _B), jnp.float32),
    name="ccbn_center",
    mesh=plsc.VectorSubcoreMesh(core_axis_name="c", subcore_axis_name="s"),
    compiler_params=pltpu.CompilerParams(
        needs_layout_passes=False,
        use_tc_tiling_on_sc=False,
        disable_bounds_checks=True,
        skip_device_barrier=True,
    ),
    scratch_types=[
        pltpu.VMEM((_BLKS_PER, _F, _B), jnp.float32),
        pltpu.VMEM((_ROWS_PER,), jnp.int32),
        pltpu.VMEM((3 * _F,), jnp.float32),
        pltpu.SemaphoreType.DMA,
    ],
)


@jax.jit
def kernel(x, labels, class_means, global_mean):
    del global_mean
    x3 = jnp.transpose(x.reshape(_NB, _B, _F), (0, 2, 1))
    out3 = _sc_call(x3, labels.astype(jnp.int32), class_means.T.reshape(3 * _F))
    return jnp.transpose(out3, (0, 2, 1)).reshape(_N, _F)


# speedup vs baseline: 1.0093x; 1.0093x over previous
import jax
import jax.numpy as jnp
from jax import lax
from jax.experimental import pallas as pl
from jax.experimental.pallas import tpu as pltpu
from jax.experimental.pallas import tpu_sc as plsc

_NC = 2
_NS = 16
_NW = _NC * _NS
_L = 16

_N = 16384
_F = 2
_B = 128
_NB = _N // _B
_BLKS_PER = _NB // _NW
_ROWS_PER = _BLKS_PER * _B
_CHUNKS = _B // _L


def _body(x_hbm, labels_hbm, cm_hbm, out_hbm, x_v, lab_v, cm_v, sem):
    wid = lax.axis_index("s") * _NC + lax.axis_index("c")
    bbase = wid * _BLKS_PER
    rbase = wid * _ROWS_PER

    cx = pltpu.async_copy(x_hbm.at[pl.ds(bbase, _BLKS_PER)], x_v, sem)
    cl = pltpu.async_copy(labels_hbm.at[pl.ds(rbase, _ROWS_PER)], lab_v, sem)
    cc = pltpu.async_copy(cm_hbm, cm_v, sem)
    cx.wait()
    cl.wait()
    cc.wait()

    for b in range(_BLKS_PER):
        for c in range(_CHUNKS):
            lab = lab_v[pl.ds(b * _B + c * _L, _L)]
            for f in range(_F):
                g = plsc.load_gather(cm_v, [lab + 3 * f if f else lab])
                x_v[b, f, pl.ds(c * _L, _L)] -= g

    pltpu.sync_copy(x_v, out_hbm.at[pl.ds(bbase, _BLKS_PER)])


_sc_call = pl.kernel(
    _body,
    out_type=jax.ShapeDtypeStruct((_NB, _F, _B), jnp.float32),
    name="ccbn_center",
    mesh=plsc.VectorSubcoreMesh(core_axis_name="c", subcore_axis_name="s"),
    compiler_params=pltpu.CompilerParams(
        needs_layout_passes=False,
        use_tc_tiling_on_sc=False,
        disable_bounds_checks=True,
    ),
    scratch_types=[
        pltpu.VMEM((_BLKS_PER, _F, _B), jnp.float32),
        pltpu.VMEM((_ROWS_PER,), jnp.int32),
        pltpu.VMEM((3 * _F,), jnp.float32),
        pltpu.SemaphoreType.DMA,
    ],
)


@jax.jit
def kernel(x, labels, class_means, global_mean):
    del global_mean
    x3 = jnp.transpose(x.reshape(_NB, _B, _F), (0, 2, 1))
    out3 = _sc_call(x3, labels.astype(jnp.int32), class_means.T.reshape(3 * _F))
    return jnp.transpose(out3, (0, 2, 1)).reshape(_N, _F)


# speedup vs baseline: 1.0302x; 1.0207x over previous
import jax
import jax.numpy as jnp
from jax import lax
from jax.experimental import pallas as pl
from jax.experimental.pallas import tpu as pltpu
from jax.experimental.pallas import tpu_sc as plsc

_NC = 2
_NS = 16
_NW = _NC * _NS
_L = 16

_N = 16384
_F = 2
_B = 128
_NB = _N // _B
_BLKS_PER = _NB // _NW
_ROWS_PER = _BLKS_PER * _B
_CHUNKS = _B // _L


def _body(x_hbm, labels_hbm, cm_hbm, out_hbm, x_v, lab_v, cm_v, sem):
    wid = lax.axis_index("s") * _NC + lax.axis_index("c")
    bbase = wid * _BLKS_PER
    rbase = wid * _ROWS_PER

    cx = pltpu.async_copy(x_hbm.at[pl.ds(bbase, _BLKS_PER)], x_v, sem)
    cl = pltpu.async_copy(labels_hbm.at[pl.ds(rbase, _ROWS_PER)], lab_v, sem)
    cc = pltpu.async_copy(cm_hbm, cm_v.at[pl.ds(0, 3 * _F)], sem)
    cx.wait()
    cl.wait()
    cc.wait()

    cmvec = cm_v[pl.ds(0, _L)]
    cmv = [
        [jnp.full((_L,), cmvec[3 * f + l], jnp.float32) for l in range(3)]
        for f in range(_F)
    ]

    for b in range(_BLKS_PER):
        for c in range(_CHUNKS):
            lab = lab_v[pl.ds(b * _B + c * _L, _L)]
            is0 = lab == 0
            is1 = lab == 1
            for f in range(_F):
                g = jnp.where(is0, cmv[f][0], jnp.where(is1, cmv[f][1], cmv[f][2]))
                x_v[b, f, pl.ds(c * _L, _L)] -= g

    pltpu.sync_copy(x_v, out_hbm.at[pl.ds(bbase, _BLKS_PER)])


_sc_call = pl.kernel(
    _body,
    out_type=jax.ShapeDtypeStruct((_NB, _F, _B), jnp.float32),
    name="ccbn_center",
    mesh=plsc.VectorSubcoreMesh(core_axis_name="c", subcore_axis_name="s"),
    compiler_params=pltpu.CompilerParams(
        needs_layout_passes=False,
        use_tc_tiling_on_sc=False,
        disable_bounds_checks=True,
    ),
    scratch_types=[
        pltpu.VMEM((_BLKS_PER, _F, _B), jnp.float32),
        pltpu.VMEM((_ROWS_PER,), jnp.int32),
        pltpu.VMEM((_L,), jnp.float32),
        pltpu.SemaphoreType.DMA,
    ],
)


@jax.jit
def kernel(x, labels, class_means, global_mean):
    del global_mean
    x3 = jnp.transpose(x.reshape(_NB, _B, _F), (0, 2, 1))
    out3 = _sc_call(x3, labels.astype(jnp.int32), class_means.T.reshape(3 * _F))
    return jnp.transpose(out3, (0, 2, 1)).reshape(_N, _F)


# speedup vs baseline: 1.0915x; 1.0595x over previous
import jax
import jax.numpy as jnp
from jax import lax
from jax.experimental import pallas as pl
from jax.experimental.pallas import tpu as pltpu
from jax.experimental.pallas import tpu_sc as plsc

_NC = 1
_NS = 16
_NW = _NC * _NS
_L = 16

_N = 16384
_F = 2
_B = 128
_NB = _N // _B
_BLKS_PER = _NB // _NW
_ROWS_PER = _BLKS_PER * _B
_CHUNKS = _B // _L


def _body(x_hbm, labels_hbm, cm_hbm, out_hbm, x_v, lab_v, cm_v, sem):
    wid = lax.axis_index("s") * _NC + lax.axis_index("c")
    bbase = wid * _BLKS_PER
    rbase = wid * _ROWS_PER

    cx = pltpu.async_copy(x_hbm.at[pl.ds(bbase, _BLKS_PER)], x_v, sem)
    cl = pltpu.async_copy(labels_hbm.at[pl.ds(rbase, _ROWS_PER)], lab_v, sem)
    cc = pltpu.async_copy(cm_hbm, cm_v.at[pl.ds(0, 3 * _F)], sem)
    cx.wait()
    cl.wait()
    cc.wait()

    cmvec = cm_v[pl.ds(0, _L)]
    cmv = [
        [jnp.full((_L,), cmvec[3 * f + l], jnp.float32) for l in range(3)]
        for f in range(_F)
    ]

    for b in range(_BLKS_PER):
        for c in range(_CHUNKS):
            lab = lab_v[pl.ds(b * _B + c * _L, _L)]
            is0 = lab == 0
            is1 = lab == 1
            for f in range(_F):
                g = jnp.where(is0, cmv[f][0], jnp.where(is1, cmv[f][1], cmv[f][2]))
                x_v[b, f, pl.ds(c * _L, _L)] -= g

    pltpu.sync_copy(x_v, out_hbm.at[pl.ds(bbase, _BLKS_PER)])


_sc_call = pl.kernel(
    _body,
    out_type=jax.ShapeDtypeStruct((_NB, _F, _B), jnp.float32),
    name="ccbn_center",
    mesh=plsc.VectorSubcoreMesh(
        core_axis_name="c", subcore_axis_name="s", num_cores=_NC
    ),
    compiler_params=pltpu.CompilerParams(
        needs_layout_passes=False,
        use_tc_tiling_on_sc=False,
        disable_bounds_checks=True,
    ),
    scratch_types=[
        pltpu.VMEM((_BLKS_PER, _F, _B), jnp.float32),
        pltpu.VMEM((_ROWS_PER,), jnp.int32),
        pltpu.VMEM((_L,), jnp.float32),
        pltpu.SemaphoreType.DMA,
    ],
)


@jax.jit
def kernel(x, labels, class_means, global_mean):
    del global_mean
    x3 = jnp.transpose(x.reshape(_NB, _B, _F), (0, 2, 1))
    out3 = _sc_call(x3, labels.astype(jnp.int32), class_means.T.reshape(3 * _F))
    return jnp.transpose(out3, (0, 2, 1)).reshape(_N, _F)


# speedup vs baseline: 1.0979x; 1.0059x over previous
import jax
import jax.numpy as jnp
from jax import lax
from jax.experimental import pallas as pl
from jax.experimental.pallas import tpu as pltpu
from jax.experimental.pallas import tpu_sc as plsc

_NC = 1
_NS = 16
_NW = _NC * _NS
_L = 16

_N = 16384
_F = 2
_B = 128
_NB = _N // _B
_BLKS_PER = _NB // _NW
_ROWS_PER = _BLKS_PER * _B
_CHUNKS = _B // _L


def _body(x_hbm, labels_hbm, cm_hbm, out_hbm, x_v, lab_v, cm_v, sem):
    wid = lax.axis_index("s") * _NC + lax.axis_index("c")
    bbase = wid * _BLKS_PER
    rbase = wid * _ROWS_PER

    cx = pltpu.async_copy(x_hbm.at[pl.ds(bbase, _BLKS_PER)], x_v, sem)
    cl = pltpu.async_copy(labels_hbm.at[pl.ds(rbase, _ROWS_PER)], lab_v, sem)
    cc = pltpu.async_copy(cm_hbm, cm_v.at[pl.ds(0, 3 * _F)], sem)
    cx.wait()
    cl.wait()
    cc.wait()

    cmvec = cm_v[pl.ds(0, _L)]
    cmv = [
        [jnp.full((_L,), cmvec[3 * f + l], jnp.float32) for l in range(3)]
        for f in range(_F)
    ]

    outs = []
    for b in range(_BLKS_PER):
        for c in range(_CHUNKS):
            lab = lab_v[pl.ds(b * _B + c * _L, _L)]
            is0 = lab == 0
            is1 = lab == 1
            for f in range(_F):
                g = jnp.where(is0, cmv[f][0], jnp.where(is1, cmv[f][1], cmv[f][2]))
                x_v[b, f, pl.ds(c * _L, _L)] -= g
        outs.append(pltpu.async_copy(x_v.at[b], out_hbm.at[bbase + b], sem))
    for h in outs:
        h.wait()


_sc_call = pl.kernel(
    _body,
    out_type=jax.ShapeDtypeStruct((_NB, _F, _B), jnp.float32),
    name="ccbn_center",
    mesh=plsc.VectorSubcoreMesh(
        core_axis_name="c", subcore_axis_name="s", num_cores=_NC
    ),
    compiler_params=pltpu.CompilerParams(
        needs_layout_passes=False,
        use_tc_tiling_on_sc=False,
        disable_bounds_checks=True,
    ),
    scratch_types=[
        pltpu.VMEM((_BLKS_PER, _F, _B), jnp.float32),
        pltpu.VMEM((_ROWS_PER,), jnp.int32),
        pltpu.VMEM((_L,), jnp.float32),
        pltpu.SemaphoreType.DMA,
    ],
)


@jax.jit
def kernel(x, labels, class_means, global_mean):
    del global_mean
    x3 = jnp.transpose(x.reshape(_NB, _B, _F), (0, 2, 1))
    out3 = _sc_call(x3, labels.astype(jnp.int32), class_means.T.reshape(3 * _F))
    return jnp.transpose(out3, (0, 2, 1)).reshape(_N, _F)
